# baseline (speedup 1.0000x reference)
import jax
import jax.numpy as jnp
from jax import lax
from jax.experimental import pallas as pl
from jax.experimental.pallas import tpu as pltpu
from jax.experimental.pallas import tpu_sc as plsc

BATCH = 4
SEQ = 2048
D = 768
NC = 2
NS = 16
NW = NC * NS
PW = SEQ // NW
CH = 32
NCHUNK = BATCH * PW // CH
LANES = 16
NJ = D // LANES
NBUF = 3


def _emb_body(ids_hbm, wte_hbm, wpe_hbm, out_hbm,
              idx_v, wpe_v, bufs, wsem, gsems, ssems):
    c = lax.axis_index("c")
    s = lax.axis_index("s")
    w = s * NC + c
    pbase = w * PW

    def gather(k):
        b, h = divmod(k, 2)
        idx = idx_v.at[b, pl.ds(h * CH, CH)]
        return pltpu.make_async_copy(wte_hbm.at[idx], bufs.at[k % NBUF],
                                     gsems.at[k % NBUF])

    def store(k):
        b, h = divmod(k, 2)
        row0 = b * SEQ + pbase + h * CH
        return pltpu.make_async_copy(bufs.at[k % NBUF],
                                     out_hbm.at[pl.ds(row0, CH)],
                                     ssems.at[k % NBUF])

    for b in range(BATCH):
        pltpu.sync_copy(ids_hbm.at[pl.ds(b * SEQ + pbase, PW)], idx_v.at[b])
    wpe_cp = pltpu.make_async_copy(wpe_hbm.at[pl.ds(pbase, PW)], wpe_v, wsem)
    wpe_cp.start()
    gather(0).start()
    gather(1).start()
    wpe_cp.wait()

    for k in range(NCHUNK):
        gather(k).wait()

        @pl.loop(0, CH)
        def _(r):
            for j in range(NJ):
                sl = pl.ds(j * LANES, LANES)
                plsc.addupdate(wpe_v.at[r, sl], wpe_v[r + CH, sl])

        store(k).start()
        if k + 2 < NCHUNK:
            if k >= 1:
                store(k - 1).wait()
            gather(k + 2).start()

    for k in range(NCHUNK - 3, NCHUNK):
        store(k).wait()


def kernel(input_ids, wte, wpe):
    ids_flat = input_ids.reshape(-1).astype(jnp.int32)
    mesh = plsc.VectorSubcoreMesh(core_axis_name="c", subcore_axis_name="s")
    run = pl.kernel(
        _emb_body,
        out_type=jax.ShapeDtypeStruct((BATCH * SEQ, D), jnp.float32),
        mesh=mesh,
        scratch_types=[
            pltpu.VMEM((BATCH, PW), jnp.int32),
            pltpu.VMEM((PW, D), jnp.float32),
            pltpu.VMEM((NBUF, CH, D), jnp.float32),
            pltpu.SemaphoreType.DMA,
            pltpu.SemaphoreType.DMA((NBUF,)),
            pltpu.SemaphoreType.DMA((NBUF,)),
        ],
    )
    out = run(ids_flat, wte, wpe)
    return out.reshape(BATCH, SEQ, D)


# speedup vs baseline: 1.0129x; 1.0129x over previous
import jax
import jax.numpy as jnp
from jax import lax
from jax.experimental import pallas as pl
from jax.experimental.pallas import tpu as pltpu
from jax.experimental.pallas import tpu_sc as plsc

BATCH = 4
SEQ = 2048
D = 768
NC = 2
NS = 16
NW = NC * NS
PW = SEQ // NW
CH = 16
NR = PW // CH
LANES = 16
NJ = D // LANES
NSET = 2


def _emb_body(ids_hbm, wte_hbm, wpe_hbm, out_hbm,
              idx_v, wbufs, bufs, wsems, gsems, ssems):
    c = lax.axis_index("c")
    s = lax.axis_index("s")
    w = s * NC + c
    pbase = w * PW

    def wpe_cp(h):
        return pltpu.make_async_copy(
            wpe_hbm.at[pl.ds(pbase + h * CH, CH)],
            wbufs.at[h % NSET], wsems.at[h % NSET])

    def gather(h, b):
        idx = idx_v.at[b, pl.ds(h * CH, CH)]
        return pltpu.make_async_copy(wte_hbm.at[idx],
                                     bufs.at[h % NSET, b],
                                     gsems.at[h % NSET, b])

    def store(h, b):
        row0 = b * SEQ + pbase + h * CH
        return pltpu.make_async_copy(bufs.at[h % NSET, b],
                                     out_hbm.at[pl.ds(row0, CH)],
                                     ssems.at[h % NSET, b])

    for b in range(BATCH):
        pltpu.sync_copy(ids_hbm.at[pl.ds(b * SEQ + pbase, PW)], idx_v.at[b])

    for h in range(NSET):
        wpe_cp(h).start()
        for b in range(BATCH):
            gather(h, b).start()

    for h in range(NR):
        st = h % NSET
        wpe_cp(h).wait()
        for b in range(BATCH):
            gather(h, b).wait()
        wbuf = wbufs.at[st]

        @pl.loop(0, CH)
        def _(r):
            wrow = [wbuf[r, pl.ds(j * LANES, LANES)] for j in range(NJ)]
            for b in range(BATCH):
                buf = bufs.at[st, b]
                for j in range(NJ):
                    plsc.addupdate(buf.at[r, pl.ds(j * LANES, LANES)],
                                   wrow[j])

        for b in range(BATCH):
            store(h, b).start()
        if h + NSET < NR:
            for b in range(BATCH):
                store(h, b).wait()
            wpe_cp(h + NSET).start()
            for b in range(BATCH):
                gather(h + NSET, b).start()

    for h in range(NR - NSET, NR):
        for b in range(BATCH):
            store(h, b).wait()


def kernel(input_ids, wte, wpe):
    ids_flat = input_ids.reshape(-1).astype(jnp.int32)
    mesh = plsc.VectorSubcoreMesh(core_axis_name="c", subcore_axis_name="s")
    run = pl.kernel(
        _emb_body,
        out_type=jax.ShapeDtypeStruct((BATCH * SEQ, D), jnp.float32),
        mesh=mesh,
        scratch_types=[
            pltpu.VMEM((BATCH, PW), jnp.int32),
            pltpu.VMEM((NSET, CH, D), jnp.float32),
            pltpu.VMEM((NSET, BATCH, CH, D), jnp.float32),
            pltpu.SemaphoreType.DMA((NSET,)),
            pltpu.SemaphoreType.DMA((NSET, BATCH)),
            pltpu.SemaphoreType.DMA((NSET, BATCH)),
        ],
    )
    out = run(ids_flat, wte, wpe)
    return out.reshape(BATCH, SEQ, D)


# speedup vs baseline: 1.0183x; 1.0054x over previous
import jax
import jax.numpy as jnp
from jax import lax
from jax.experimental import pallas as pl
from jax.experimental.pallas import tpu as pltpu
from jax.experimental.pallas import tpu_sc as plsc

BATCH = 4
SEQ = 2048
D = 768
NC = 2
NS = 16
NW = NC * NS
PW = SEQ // NW
CH = 16
NR = PW // CH
LANES = 16
NJ = D // LANES
NSET = 2


def _emb_body(ids_hbm, wte_hbm, wpe_hbm, out_hbm,
              idx_v, wbufs, bufs, wsems, gsems, ssems):
    c = lax.axis_index("c")
    s = lax.axis_index("s")
    w = s * NC + c
    pbase = w * PW

    def wpe_cp(h):
        return pltpu.make_async_copy(
            wpe_hbm.at[pl.ds(pbase + h * CH, CH)],
            wbufs.at[h % NSET], wsems.at[h % NSET])

    def gather(h, b):
        idx = idx_v.at[b, pl.ds(h * CH, CH)]
        return pltpu.make_async_copy(wte_hbm.at[idx],
                                     bufs.at[h % NSET, b],
                                     gsems.at[h % NSET, b])

    def store(h, b):
        row0 = pbase + h * CH
        return pltpu.make_async_copy(bufs.at[h % NSET, b],
                                     out_hbm.at[b, pl.ds(row0, CH)],
                                     ssems.at[h % NSET, b])

    for b in range(BATCH):
        pltpu.sync_copy(ids_hbm.at[b, pl.ds(pbase, PW)], idx_v.at[b])

    for h in range(NSET):
        wpe_cp(h).start()
        for b in range(BATCH):
            gather(h, b).start()

    for h in range(NR):
        st = h % NSET
        wpe_cp(h).wait()
        for b in range(BATCH):
            gather(h, b).wait()
        wbuf = wbufs.at[st]

        @pl.loop(0, CH)
        def _(r):
            wrow = [wbuf[r, pl.ds(j * LANES, LANES)] for j in range(NJ)]
            for b in range(BATCH):
                buf = bufs.at[st, b]
                for j in range(NJ):
                    plsc.addupdate(buf.at[r, pl.ds(j * LANES, LANES)],
                                   wrow[j])

        for b in range(BATCH):
            store(h, b).start()
        if h + NSET < NR:
            for b in range(BATCH):
                store(h, b).wait()
            wpe_cp(h + NSET).start()
            for b in range(BATCH):
                gather(h + NSET, b).start()

    for h in range(NR - NSET, NR):
        for b in range(BATCH):
            store(h, b).wait()


def kernel(input_ids, wte, wpe):
    mesh = plsc.VectorSubcoreMesh(core_axis_name="c", subcore_axis_name="s")
    run = pl.kernel(
        _emb_body,
        out_type=jax.ShapeDtypeStruct((BATCH, SEQ, D), jnp.float32),
        mesh=mesh,
        scratch_types=[
            pltpu.VMEM((BATCH, PW), jnp.int32),
            pltpu.VMEM((NSET, CH, D), jnp.float32),
            pltpu.VMEM((NSET, BATCH, CH, D), jnp.float32),
            pltpu.SemaphoreType.DMA((NSET,)),
            pltpu.SemaphoreType.DMA((NSET, BATCH)),
            pltpu.SemaphoreType.DMA((NSET, BATCH)),
        ],
    )
    return run(input_ids, wte, wpe)
